# baseline (speedup 1.0000x reference)
import functools

import jax
import jax.numpy as jnp
from jax import lax
from jax.experimental import pallas as pl
from jax.experimental.pallas import tpu as pltpu
from jax.experimental.pallas import tpu_sc as plsc

L = 16
NW = 32
CHUNK = 2608


_GATHER_DNUMS = lax.GatherDimensionNumbers(
    offset_dims=(), collapsed_slice_dims=(0,), start_index_map=(0,))


def _lane_take(v, idx):
    return lax.gather(v, idx[:, None], _GATHER_DNUMS, (1,),
                      mode=lax.GatherScatterMode.PROMISE_IN_BOUNDS)


def _union_body(n_points, n_chunks, x_hbm, wd_hbm, b_hbm, wa_hbm,
                minv_hbm, sel_hbm,
                wd_v, b_v, wa_v,
                xa, mva, sela,
                xb, mvb, selb,
                sin_a, sin_b, sout_a, sout_b):
    info = plsc.get_sparse_core_info()
    nc = info.num_cores
    wid = lax.axis_index("s") * nc + lax.axis_index("c")
    span = n_chunks * CHUNK

    pltpu.sync_copy(wd_hbm, wd_v)
    pltpu.sync_copy(b_hbm, b_v)
    pltpu.sync_copy(wa_hbm, wa_v)

    wcol = [wd_v[pl.ds(16 * d, 16)] for d in range(3)]
    bvec = b_v[...]
    wd = [[wcol[d][k] for d in range(3)] for k in range(16)]
    bs = [bvec[k] for k in range(16)]

    iota = lax.iota(jnp.int32, L)
    deint = []
    for d in range(3):
        pos = iota * 3 + d
        deint.append((pos & 15, pos >> 4))

    groups = CHUNK // L

    def cstart(c):
        s = jnp.minimum(wid * span + c * CHUNK, n_points - CHUNK)
        return pl.multiple_of(s, 8)

    def fire_in(c, xv, sem):
        s = cstart(c)
        pltpu.async_copy(x_hbm.at[pl.ds(s * 3, CHUNK * 3)], xv, sem)

    def wait_in(xv, sem):
        pltpu.make_async_copy(x_hbm.at[pl.ds(0, CHUNK * 3)], xv, sem).wait()

    def fire_out(c, mv, sel, sem):
        s = cstart(c)
        pltpu.async_copy(mv, minv_hbm.at[pl.ds(s, CHUNK)], sem)
        pltpu.async_copy(sel, sel_hbm.at[pl.ds(s * 16, CHUNK * 16)], sem)

    def wait_out(mv, sel, sem):
        pltpu.make_async_copy(mv, minv_hbm.at[pl.ds(0, CHUNK)], sem).wait()
        pltpu.make_async_copy(
            sel, sel_hbm.at[pl.ds(0, CHUNK * 16)], sem).wait()

    def compute(xv, mv, sel):
        def group_body(g, gcarry):
            gb = pl.multiple_of(g * L, 8)
            v = [xv[pl.ds(gb * 3 + 16 * i, L)] for i in range(3)]
            planes = []
            for d in range(3):
                idx, src = deint[d]
                gsh = [_lane_take(v[i], idx) for i in range(3)]
                planes.append(jnp.where(src == 0, gsh[0],
                                        jnp.where(src == 1, gsh[1], gsh[2])))
            x0, x1, x2 = planes

            minv = x0 * wd[0][0] + x1 * wd[0][1] + x2 * wd[0][2] + bs[0]
            idxv = jnp.zeros((L,), jnp.int32)
            for k in range(1, 16):
                t = x0 * wd[k][0] + x1 * wd[k][1] + x2 * wd[k][2] + bs[k]
                m = t < minv
                idxv = jnp.where(m, k, idxv)
                minv = jnp.where(m, t, minv)
            mv[pl.ds(gb, L)] = minv

            for a in range(16):
                g0 = _lane_take(wa_v[pl.ds(a * 16, L)], idxv)
                g1 = _lane_take(wa_v[pl.ds(256 + a * 16, L)], idxv)
                g2 = _lane_take(wa_v[pl.ds(512 + a * 16, L)], idxv)
                sv = x0 * g0 + x1 * g1 + x2 * g2
                ob = pl.multiple_of((gb + a) * 16, 16)
                sel[pl.ds(ob, L)] = sv
            return gcarry

        lax.fori_loop(0, groups, group_body, 0, unroll=False)

    fire_in(0, xa, sin_a)

    def body2(c2, carry):
        c = 2 * c2
        wait_in(xa, sin_a)

        @pl.when(c + 1 < n_chunks)
        def _():
            fire_in(c + 1, xb, sin_b)

        @pl.when(c2 >= 1)
        def _():
            wait_out(mva, sela, sout_a)

        compute(xa, mva, sela)
        fire_out(c, mva, sela, sout_a)

        wait_in(xb, sin_b)

        @pl.when(c + 2 < n_chunks)
        def _():
            fire_in(c + 2, xa, sin_a)

        @pl.when(c2 >= 1)
        def _():
            wait_out(mvb, selb, sout_b)

        compute(xb, mvb, selb)
        fire_out(c + 1, mvb, selb, sout_b)
        return carry

    lax.fori_loop(0, n_chunks // 2, body2, 0, unroll=False)
    wait_out(mva, sela, sout_a)
    wait_out(mvb, selb, sout_b)


def kernel(x, W_d, b_d, W_a):
    n, d = x.shape
    k = W_d.shape[0]
    a = W_a.shape[2]
    assert (d, k, a) == (3, 16, 16)
    n_chunks = -(-n // (NW * CHUNK))
    n_chunks += n_chunks % 2

    body = functools.partial(_union_body, n, n_chunks)
    run = pl.kernel(
        body,
        out_type=(
            jax.ShapeDtypeStruct((n,), jnp.float32),
            jax.ShapeDtypeStruct((n * 16,), jnp.float32),
        ),
        mesh=plsc.VectorSubcoreMesh(core_axis_name="c", subcore_axis_name="s"),
        scratch_types=[
            pltpu.VMEM((48,), jnp.float32),
            pltpu.VMEM((16,), jnp.float32),
            pltpu.VMEM((768,), jnp.float32),
            pltpu.VMEM((CHUNK * 3,), jnp.float32),
            pltpu.VMEM((CHUNK,), jnp.float32),
            pltpu.VMEM((CHUNK * 16,), jnp.float32),
            pltpu.VMEM((CHUNK * 3,), jnp.float32),
            pltpu.VMEM((CHUNK,), jnp.float32),
            pltpu.VMEM((CHUNK * 16,), jnp.float32),
            pltpu.SemaphoreType.DMA,
            pltpu.SemaphoreType.DMA,
            pltpu.SemaphoreType.DMA,
            pltpu.SemaphoreType.DMA,
        ],
    )
    def _bf16_round(v):
        u = lax.bitcast_convert_type(v, jnp.uint32)
        r = (u + jnp.uint32(0x7FFF) + ((u >> 16) & jnp.uint32(1))) \
            & jnp.uint32(0xFFFF0000)
        return lax.bitcast_convert_type(r, jnp.float32)

    xb = _bf16_round(x)
    wdb = _bf16_round(W_d)
    wab = _bf16_round(W_a)
    min_vals, sel_flat = run(
        xb.reshape(-1),
        wdb.T.reshape(-1),
        b_d,
        wab.transpose(1, 2, 0).reshape(-1),
    )
    sel = sel_flat.reshape(n // 16, 16, 16).swapaxes(1, 2).reshape(n, 16)
    return min_vals, sel


# speedup vs baseline: 1.0881x; 1.0881x over previous
import functools

import jax
import jax.numpy as jnp
from jax import lax
from jax.experimental import pallas as pl
from jax.experimental.pallas import tpu as pltpu
from jax.experimental.pallas import tpu_sc as plsc

L = 16
NW = 32
CHUNK = 2608


_GATHER_DNUMS = lax.GatherDimensionNumbers(
    offset_dims=(), collapsed_slice_dims=(0,), start_index_map=(0,))


def _lane_take(v, idx):
    return lax.gather(v, idx[:, None], _GATHER_DNUMS, (1,),
                      mode=lax.GatherScatterMode.PROMISE_IN_BOUNDS)


def _union_body(n_points, n_chunks, x_hbm, wd_hbm, b_hbm, wa_hbm,
                minv_hbm, sel_hbm,
                wd_v, b_v, wa_v,
                xa, mva, sela,
                xb, mvb, selb,
                sin_a, sin_b, sout_a, sout_b):
    info = plsc.get_sparse_core_info()
    nc = info.num_cores
    wid = lax.axis_index("s") * nc + lax.axis_index("c")
    span = n_chunks * CHUNK

    pltpu.sync_copy(wd_hbm, wd_v)
    pltpu.sync_copy(b_hbm, b_v)
    pltpu.sync_copy(wa_hbm, wa_v)

    wcol = [wd_v[pl.ds(16 * d, 16)] for d in range(3)]
    bvec = b_v[...]
    wd = [[wcol[d][k] for d in range(3)] for k in range(16)]
    bs = [bvec[k] for k in range(16)]

    iota = lax.iota(jnp.int32, L)
    deint = []
    for d in range(3):
        pos = iota * 3 + d
        deint.append((pos & 15, pos >> 4))

    groups = CHUNK // L

    def cstart(c):
        s = jnp.minimum(wid * span + c * CHUNK, n_points - CHUNK)
        return pl.multiple_of(s, 8)

    def fire_in(c, xv, sem):
        s = cstart(c)
        pltpu.async_copy(x_hbm.at[pl.ds(s * 3, CHUNK * 3)], xv, sem)

    def wait_in(xv, sem):
        pltpu.make_async_copy(x_hbm.at[pl.ds(0, CHUNK * 3)], xv, sem).wait()

    def fire_out(c, mv, sel, sem):
        s = cstart(c)
        pltpu.async_copy(mv, minv_hbm.at[pl.ds(s, CHUNK)], sem)
        pltpu.async_copy(sel, sel_hbm.at[pl.ds(s * 16, CHUNK * 16)], sem)

    def wait_out(mv, sel, sem):
        pltpu.make_async_copy(mv, minv_hbm.at[pl.ds(0, CHUNK)], sem).wait()
        pltpu.make_async_copy(
            sel, sel_hbm.at[pl.ds(0, CHUNK * 16)], sem).wait()

    def compute(xv, mv, sel):
        def group_body(g, gcarry):
            gb = pl.multiple_of(g * L, 8)
            v = [xv[pl.ds(gb * 3 + 16 * i, L)] for i in range(3)]
            planes = []
            for d in range(3):
                idx, src = deint[d]
                gsh = [_lane_take(v[i], idx) for i in range(3)]
                planes.append(jnp.where(src == 0, gsh[0],
                                        jnp.where(src == 1, gsh[1], gsh[2])))
            x0, x1, x2 = planes

            minv = x0 * wd[0][0] + x1 * wd[0][1] + x2 * wd[0][2] + bs[0]
            idxv = jnp.zeros((L,), jnp.int32)
            for k in range(1, 16):
                t = x0 * wd[k][0] + x1 * wd[k][1] + x2 * wd[k][2] + bs[k]
                m = t < minv
                idxv = jnp.where(m, k, idxv)
                minv = jnp.where(m, t, minv)
            mv[pl.ds(gb, L)] = minv

            cols = []
            for a in range(16):
                g0 = _lane_take(wa_v[pl.ds(a * 16, L)], idxv)
                g1 = _lane_take(wa_v[pl.ds(256 + a * 16, L)], idxv)
                g2 = _lane_take(wa_v[pl.ds(512 + a * 16, L)], idxv)
                cols.append(x0 * g0 + x1 * g1 + x2 * g2)

            for st in range(4):
                b = 1 << st
                bit0 = ((iota >> st) & 1) == 0
                nxt = [None] * 16
                for i in range(16):
                    if i & b:
                        continue
                    va, vb = cols[i], cols[i + b]
                    nxt[i] = jnp.where(bit0, va, _lane_take(vb, iota ^ b))
                    nxt[i + b] = jnp.where(bit0, _lane_take(va, iota ^ b), vb)
                cols = nxt
            for p in range(16):
                ob = pl.multiple_of((gb + p) * 16, 16)
                sel[pl.ds(ob, L)] = cols[p]
            return gcarry

        lax.fori_loop(0, groups, group_body, 0, unroll=False)

    fire_in(0, xa, sin_a)

    def body2(c2, carry):
        c = 2 * c2
        wait_in(xa, sin_a)

        @pl.when(c + 1 < n_chunks)
        def _():
            fire_in(c + 1, xb, sin_b)

        @pl.when(c2 >= 1)
        def _():
            wait_out(mva, sela, sout_a)

        compute(xa, mva, sela)
        fire_out(c, mva, sela, sout_a)

        wait_in(xb, sin_b)

        @pl.when(c + 2 < n_chunks)
        def _():
            fire_in(c + 2, xa, sin_a)

        @pl.when(c2 >= 1)
        def _():
            wait_out(mvb, selb, sout_b)

        compute(xb, mvb, selb)
        fire_out(c + 1, mvb, selb, sout_b)
        return carry

    lax.fori_loop(0, n_chunks // 2, body2, 0, unroll=False)
    wait_out(mva, sela, sout_a)
    wait_out(mvb, selb, sout_b)


def kernel(x, W_d, b_d, W_a):
    n, d = x.shape
    k = W_d.shape[0]
    a = W_a.shape[2]
    assert (d, k, a) == (3, 16, 16)
    n_chunks = -(-n // (NW * CHUNK))
    n_chunks += n_chunks % 2

    body = functools.partial(_union_body, n, n_chunks)
    run = pl.kernel(
        body,
        out_type=(
            jax.ShapeDtypeStruct((n,), jnp.float32),
            jax.ShapeDtypeStruct((n * 16,), jnp.float32),
        ),
        mesh=plsc.VectorSubcoreMesh(core_axis_name="c", subcore_axis_name="s"),
        scratch_types=[
            pltpu.VMEM((48,), jnp.float32),
            pltpu.VMEM((16,), jnp.float32),
            pltpu.VMEM((768,), jnp.float32),
            pltpu.VMEM((CHUNK * 3,), jnp.float32),
            pltpu.VMEM((CHUNK,), jnp.float32),
            pltpu.VMEM((CHUNK * 16,), jnp.float32),
            pltpu.VMEM((CHUNK * 3,), jnp.float32),
            pltpu.VMEM((CHUNK,), jnp.float32),
            pltpu.VMEM((CHUNK * 16,), jnp.float32),
            pltpu.SemaphoreType.DMA,
            pltpu.SemaphoreType.DMA,
            pltpu.SemaphoreType.DMA,
            pltpu.SemaphoreType.DMA,
        ],
    )
    def _bf16_round(v):
        u = lax.bitcast_convert_type(v, jnp.uint32)
        r = (u + jnp.uint32(0x7FFF) + ((u >> 16) & jnp.uint32(1))) \
            & jnp.uint32(0xFFFF0000)
        return lax.bitcast_convert_type(r, jnp.float32)

    xb = _bf16_round(x)
    wdb = _bf16_round(W_d)
    wab = _bf16_round(W_a)
    min_vals, sel_flat = run(
        xb.reshape(-1),
        wdb.T.reshape(-1),
        b_d,
        wab.transpose(1, 2, 0).reshape(-1),
    )
    return min_vals, sel_flat.reshape(n, 16)


# speedup vs baseline: 5.0669x; 4.6569x over previous
import functools

import jax
import jax.numpy as jnp
from jax import lax
from jax.experimental import pallas as pl
from jax.experimental.pallas import tpu as pltpu
from jax.experimental.pallas import tpu_sc as plsc

L = 16
NW = 32
CHUNK = 2608


def _union_body(n_points, n_chunks, x_hbm, wd_hbm, b_hbm, wa_hbm,
                minv_hbm, sel_hbm,
                wd_v, b_v, wa_v,
                xa0, xa1, xa2, mva, sela,
                xb0, xb1, xb2, mvb, selb,
                sin_a, sin_b, sout_a, sout_b):
    info = plsc.get_sparse_core_info()
    nc = info.num_cores
    wid = lax.axis_index("s") * nc + lax.axis_index("c")
    span = n_chunks * CHUNK

    pltpu.sync_copy(wd_hbm, wd_v)
    pltpu.sync_copy(b_hbm, b_v)
    pltpu.sync_copy(wa_hbm, wa_v)

    wcol = [wd_v[pl.ds(16 * d, 16)] for d in range(3)]
    bvec = b_v[...]
    wd = [[wcol[d][k] for d in range(3)] for k in range(16)]
    bs = [bvec[k] for k in range(16)]

    groups = CHUNK // L

    def cstart(c):
        s = jnp.minimum(wid * span + c * CHUNK, n_points - CHUNK)
        return pl.multiple_of(s, 8)

    def fire_in(c, bufs, sem):
        s = cstart(c)
        for d, dst in enumerate(bufs):
            pltpu.async_copy(x_hbm.at[pl.ds(d * n_points + s, CHUNK)],
                             dst, sem)

    def wait_in(bufs, sem):
        for dst in bufs:
            pltpu.make_async_copy(x_hbm.at[pl.ds(0, CHUNK)], dst, sem).wait()

    def fire_out(c, mv, sel, sem):
        s = cstart(c)
        pltpu.async_copy(mv, minv_hbm.at[pl.ds(s, CHUNK)], sem)
        pltpu.async_copy(sel, sel_hbm.at[pl.ds(s * 16, CHUNK * 16)], sem)

    def wait_out(mv, sel, sem):
        pltpu.make_async_copy(mv, minv_hbm.at[pl.ds(0, CHUNK)], sem).wait()
        pltpu.make_async_copy(
            sel, sel_hbm.at[pl.ds(0, CHUNK * 16)], sem).wait()

    def compute(x0b, x1b, x2b, mv, sel):
        def group_body(g, gcarry):
            gb = pl.multiple_of(g * L, 8)
            x0 = x0b[pl.ds(gb, L)]
            x1 = x1b[pl.ds(gb, L)]
            x2 = x2b[pl.ds(gb, L)]

            minv = x0 * wd[0][0] + x1 * wd[0][1] + x2 * wd[0][2] + bs[0]
            idxv = jnp.zeros((L,), jnp.int32)
            for k in range(1, 16):
                t = x0 * wd[k][0] + x1 * wd[k][1] + x2 * wd[k][2] + bs[k]
                m = t < minv
                idxv = jnp.where(m, k, idxv)
                minv = jnp.where(m, t, minv)
            mv[pl.ds(gb, L)] = minv

            base = idxv * 48
            for p in range(L):
                jb = pl.multiple_of(base[p], 16)
                w0 = wa_v[pl.ds(jb, L)]
                w1 = wa_v[pl.ds(jb + 16, L)]
                w2 = wa_v[pl.ds(jb + 32, L)]
                sv = x0[p] * w0 + x1[p] * w1 + x2[p] * w2
                ob = pl.multiple_of((gb + p) * 16, 16)
                sel[pl.ds(ob, L)] = sv
            return gcarry

        lax.fori_loop(0, groups, group_body, 0, unroll=False)

    bufa = (xa0, xa1, xa2)
    bufb = (xb0, xb1, xb2)
    fire_in(0, bufa, sin_a)

    def body2(c2, carry):
        c = 2 * c2
        wait_in(bufa, sin_a)

        @pl.when(c + 1 < n_chunks)
        def _():
            fire_in(c + 1, bufb, sin_b)

        @pl.when(c2 >= 1)
        def _():
            wait_out(mva, sela, sout_a)

        compute(xa0, xa1, xa2, mva, sela)
        fire_out(c, mva, sela, sout_a)

        wait_in(bufb, sin_b)

        @pl.when(c + 2 < n_chunks)
        def _():
            fire_in(c + 2, bufa, sin_a)

        @pl.when(c2 >= 1)
        def _():
            wait_out(mvb, selb, sout_b)

        compute(xb0, xb1, xb2, mvb, selb)
        fire_out(c + 1, mvb, selb, sout_b)
        return carry

    lax.fori_loop(0, n_chunks // 2, body2, 0, unroll=False)
    wait_out(mva, sela, sout_a)
    wait_out(mvb, selb, sout_b)


def kernel(x, W_d, b_d, W_a):
    n, d = x.shape
    k = W_d.shape[0]
    a = W_a.shape[2]
    assert (d, k, a) == (3, 16, 16)
    n_chunks = -(-n // (NW * CHUNK))
    n_chunks += n_chunks % 2

    body = functools.partial(_union_body, n, n_chunks)
    xbuf = lambda: pltpu.VMEM((CHUNK,), jnp.float32)
    run = pl.kernel(
        body,
        out_type=(
            jax.ShapeDtypeStruct((n,), jnp.float32),
            jax.ShapeDtypeStruct((n * 16,), jnp.float32),
        ),
        mesh=plsc.VectorSubcoreMesh(core_axis_name="c", subcore_axis_name="s"),
        scratch_types=[
            pltpu.VMEM((48,), jnp.float32),
            pltpu.VMEM((16,), jnp.float32),
            pltpu.VMEM((768,), jnp.float32),
            xbuf(), xbuf(), xbuf(),
            pltpu.VMEM((CHUNK,), jnp.float32),
            pltpu.VMEM((CHUNK * 16,), jnp.float32),
            xbuf(), xbuf(), xbuf(),
            pltpu.VMEM((CHUNK,), jnp.float32),
            pltpu.VMEM((CHUNK * 16,), jnp.float32),
            pltpu.SemaphoreType.DMA,
            pltpu.SemaphoreType.DMA,
            pltpu.SemaphoreType.DMA,
            pltpu.SemaphoreType.DMA,
        ],
    )
    def _bf16_round(v):
        u = lax.bitcast_convert_type(v, jnp.uint32)
        r = (u + jnp.uint32(0x7FFF) + ((u >> 16) & jnp.uint32(1))) \
            & jnp.uint32(0xFFFF0000)
        return lax.bitcast_convert_type(r, jnp.float32)

    xb = _bf16_round(x)
    wdb = _bf16_round(W_d)
    wab = _bf16_round(W_a)
    min_vals, sel_flat = run(
        xb.T.reshape(-1),
        wdb.T.reshape(-1),
        b_d,
        wab.reshape(-1),
    )
    return min_vals, sel_flat.reshape(n, 16)
